# Initial kernel scaffold; baseline (speedup 1.0000x reference)
import jax
import jax.numpy as jnp
from jax.experimental import pallas as pl


def kernel(conv1_w, conv1_b, conv2_w, conv2_b, fc1_w, fc1_b, fc2_w, fc2_b, fc3_w, fc3_b, x):
    raise NotImplementedError("write your pallas kernel here")


# speedup vs baseline: 12.8946x; 12.8946x over previous
import jax
import jax.numpy as jnp
from jax.experimental import pallas as pl
from jax.experimental.pallas import tpu as pltpu

_TN = 128


def _lenet_kernel(x_ref, w1_ref, b1_ref, w2_ref, b2_ref,
                  fc1_ref, fb1_ref, fc2_ref, fb2_ref, fc3_ref, fb3_ref,
                  o_ref, y1t_ref):
    tn = x_ref.shape[1]
    xb = x_ref[...].reshape(28, 28, tn)

    accs = [None] * 16
    for i in range(5):
        for j in range(5):
            xs = xb[i:i + 24, j:j + 24, :]
            for co in range(16):
                w = w1_ref[co, i * 5 + j]
                accs[co] = w * xs if accs[co] is None else accs[co] + w * xs
    pooled = []
    for co in range(16):
        m = jnp.maximum(accs[co] + b1_ref[co], 0.0)
        r = m.reshape(12, 2, 24, tn)
        m = jnp.maximum(r[:, 0], r[:, 1])
        c = m.reshape(12, 12, 2, tn)
        pooled.append(jnp.maximum(c[:, :, 0], c[:, :, 1]))

    y1 = jnp.stack(pooled, axis=2)
    y1t_ref[...] = y1.reshape(12 * 12 * 16, tn).T.reshape(tn, 12, 12, 16)

    h = None
    for yp in range(5):
        zrows = []
        for dy in range(2):
            y = 2 * yp + dy
            acc2 = None
            for t in range(9):
                i, j = t // 3, t % 3
                piece = y1t_ref[:, y + i, j:j + 10, :].reshape(tn * 10, 16)
                d = jnp.dot(piece, w2_ref[t],
                            preferred_element_type=jnp.float32)
                acc2 = d if acc2 is None else acc2 + d
            z = jnp.maximum(acc2 + b2_ref[...], 0.0)
            zrows.append(z.reshape(tn, 10, 32))
        m = jnp.maximum(zrows[0], zrows[1])
        c = m.reshape(tn, 5, 2, 32)
        p = jnp.maximum(c[:, :, 0], c[:, :, 1])
        for xp in range(5):
            d = jnp.dot(p[:, xp, :], fc1_ref[yp * 5 + xp],
                        preferred_element_type=jnp.float32)
            h = d if h is None else h + d
    h = jnp.maximum(h + fb1_ref[...], 0.0)
    h = jnp.dot(h, fc2_ref[...], preferred_element_type=jnp.float32)
    h = jnp.maximum(h + fb2_ref[...], 0.0)
    o_ref[...] = (jnp.dot(h, fc3_ref[...], preferred_element_type=jnp.float32)
                  + fb3_ref[...])


def kernel(conv1_w, conv1_b, conv2_w, conv2_b,
           fc1_w, fc1_b, fc2_w, fc2_b, fc3_w, fc3_b, x):
    N = x.shape[0]
    x2 = x.reshape(N, 28 * 28).T
    n_pad = -(-N // _TN) * _TN
    if n_pad != N:
        x2 = jnp.pad(x2, ((0, 0), (0, n_pad - N)))

    w1s = conv1_w.reshape(16, 25)
    w2t = conv2_w.transpose(2, 3, 1, 0).reshape(9, 16, 32)
    fc1_wc = fc1_w.reshape(120, 32, 5, 5).transpose(2, 3, 1, 0).reshape(25, 32, 120)

    rep2 = lambda i: (0, 0)
    rep3 = lambda i: (0, 0, 0)
    out = pl.pallas_call(
        _lenet_kernel,
        out_shape=jax.ShapeDtypeStruct((n_pad, 10), jnp.float32),
        grid=(n_pad // _TN,),
        in_specs=[
            pl.BlockSpec((28 * 28, _TN), lambda i: (0, i)),
            pl.BlockSpec(memory_space=pltpu.SMEM),
            pl.BlockSpec(memory_space=pltpu.SMEM),
            pl.BlockSpec((9, 16, 32), rep3),
            pl.BlockSpec((1, 32), rep2),
            pl.BlockSpec((25, 32, 120), rep3),
            pl.BlockSpec((1, 120), rep2),
            pl.BlockSpec((120, 84), rep2),
            pl.BlockSpec((1, 84), rep2),
            pl.BlockSpec((84, 10), rep2),
            pl.BlockSpec((1, 10), rep2),
        ],
        out_specs=pl.BlockSpec((_TN, 10), lambda i: (i, 0)),
        scratch_shapes=[pltpu.VMEM((_TN, 12, 12, 16), jnp.float32)],
        compiler_params=pltpu.CompilerParams(
            dimension_semantics=("parallel",),
            vmem_limit_bytes=48 * 1024 * 1024,
        ),
    )(x2, w1s, conv1_b, w2t, conv2_b.reshape(1, 32),
      fc1_wc, fc1_b.reshape(1, 120), fc2_w.T, fc2_b.reshape(1, 84),
      fc3_w.T, fc3_b.reshape(1, 10))
    return out[:N]


# speedup vs baseline: 20.4392x; 1.5851x over previous
import jax
import jax.numpy as jnp
from jax.experimental import pallas as pl
from jax.experimental.pallas import tpu as pltpu

_TN = 256


def _lenet_kernel(x_ref, w1_ref, b1_ref, w2_ref, b2_ref,
                  fc1_ref, fb1_ref, fc2_ref, fb2_ref, fc3_ref, fb3_ref,
                  o_ref, y1t_ref):
    tn = x_ref.shape[1]
    xb = x_ref[...].reshape(28, 28, tn)

    accs = [None] * 16
    for i in range(5):
        for j in range(5):
            xs = xb[i:i + 24, j:j + 24, :]
            for co in range(16):
                w = w1_ref[co, i * 5 + j]
                accs[co] = w * xs if accs[co] is None else accs[co] + w * xs
    pooled = []
    for co in range(16):
        m = jnp.maximum(accs[co] + b1_ref[co], 0.0)
        r = m.reshape(12, 2, 24, tn)
        m = jnp.maximum(r[:, 0], r[:, 1])
        c = m.reshape(12, 12, 2, tn)
        pooled.append(jnp.maximum(c[:, :, 0], c[:, :, 1]))

    y1 = jnp.stack(pooled, axis=2)
    y1t_ref[...] = y1.reshape(12 * 12 * 16, tn).T.reshape(tn, 12, 12, 16)

    h = None
    for yp in range(5):
        zrows = []
        for dy in range(2):
            y = 2 * yp + dy
            acc2 = None
            for t in range(9):
                i, j = t // 3, t % 3
                piece = y1t_ref[:, y + i, j:j + 10, :].reshape(tn * 10, 16)
                d = jnp.dot(piece, w2_ref[t],
                            preferred_element_type=jnp.float32)
                acc2 = d if acc2 is None else acc2 + d
            z = jnp.maximum(acc2 + b2_ref[...], 0.0)
            zrows.append(z.reshape(tn, 10, 32))
        m = jnp.maximum(zrows[0], zrows[1])
        c = m.reshape(tn, 5, 2, 32)
        p = jnp.maximum(c[:, :, 0], c[:, :, 1])
        for xp in range(5):
            d = jnp.dot(p[:, xp, :], fc1_ref[yp * 5 + xp],
                        preferred_element_type=jnp.float32)
            h = d if h is None else h + d
    h = jnp.maximum(h + fb1_ref[...], 0.0)
    h = jnp.dot(h, fc2_ref[...], preferred_element_type=jnp.float32)
    h = jnp.maximum(h + fb2_ref[...], 0.0)
    o_ref[...] = (jnp.dot(h, fc3_ref[...], preferred_element_type=jnp.float32)
                  + fb3_ref[...])


def kernel(conv1_w, conv1_b, conv2_w, conv2_b,
           fc1_w, fc1_b, fc2_w, fc2_b, fc3_w, fc3_b, x):
    N = x.shape[0]
    x2 = x.reshape(N, 28 * 28).T
    n_pad = -(-N // _TN) * _TN
    if n_pad != N:
        x2 = jnp.pad(x2, ((0, 0), (0, n_pad - N)))

    w1s = conv1_w.reshape(16, 25)
    w2t = conv2_w.transpose(2, 3, 1, 0).reshape(9, 16, 32)
    fc1_wc = fc1_w.reshape(120, 32, 5, 5).transpose(2, 3, 1, 0).reshape(25, 32, 120)

    rep2 = lambda i: (0, 0)
    rep3 = lambda i: (0, 0, 0)
    out = pl.pallas_call(
        _lenet_kernel,
        out_shape=jax.ShapeDtypeStruct((n_pad, 10), jnp.float32),
        grid=(n_pad // _TN,),
        in_specs=[
            pl.BlockSpec((28 * 28, _TN), lambda i: (0, i)),
            pl.BlockSpec(memory_space=pltpu.SMEM),
            pl.BlockSpec(memory_space=pltpu.SMEM),
            pl.BlockSpec((9, 16, 32), rep3),
            pl.BlockSpec((1, 32), rep2),
            pl.BlockSpec((25, 32, 120), rep3),
            pl.BlockSpec((1, 120), rep2),
            pl.BlockSpec((120, 84), rep2),
            pl.BlockSpec((1, 84), rep2),
            pl.BlockSpec((84, 10), rep2),
            pl.BlockSpec((1, 10), rep2),
        ],
        out_specs=pl.BlockSpec((_TN, 10), lambda i: (i, 0)),
        scratch_shapes=[pltpu.VMEM((_TN, 12, 12, 16), jnp.float32)],
        compiler_params=pltpu.CompilerParams(
            dimension_semantics=("parallel",),
            vmem_limit_bytes=48 * 1024 * 1024,
        ),
    )(x2, w1s, conv1_b, w2t, conv2_b.reshape(1, 32),
      fc1_wc, fc1_b.reshape(1, 120), fc2_w.T, fc2_b.reshape(1, 84),
      fc3_w.T, fc3_b.reshape(1, 10))
    return out[:N]


# speedup vs baseline: 20.8964x; 1.0224x over previous
import jax
import jax.numpy as jnp
from jax.experimental import pallas as pl
from jax.experimental.pallas import tpu as pltpu

_TN = 256


def _lenet_kernel(x_ref, w1_ref, b1_ref, w2_ref, b2_ref,
                  fc1_ref, fb1_ref, fc2_ref, fb2_ref, fc3_ref, fb3_ref,
                  o_ref, y1_ref):
    tn = x_ref.shape[1]
    xb = x_ref[...].reshape(28, 28, tn)

    accs = [None] * 16
    for i in range(5):
        for j in range(5):
            xs = xb[i:i + 24, j:j + 24, :]
            for co in range(16):
                w = w1_ref[co, i * 5 + j]
                accs[co] = w * xs if accs[co] is None else accs[co] + w * xs
    for co in range(16):
        m = jnp.maximum(accs[co] + b1_ref[co], 0.0)
        r = m.reshape(12, 2, 24, tn)
        m = jnp.maximum(r[:, 0], r[:, 1])
        c = m.reshape(12, 12, 2, tn)
        y1_ref[co] = jnp.maximum(c[:, :, 0], c[:, :, 1])

    w2 = w2_ref[...]
    pooled2 = []
    for yp in range(5):
        for xp in range(5):
            m = None
            for a in range(2):
                for b in range(2):
                    y, x = 2 * yp + a, 2 * xp + b
                    patch = y1_ref[:, y:y + 3, x:x + 3, :].reshape(144, tn)
                    d = jnp.dot(w2, patch, preferred_element_type=jnp.float32)
                    m = d if m is None else jnp.maximum(m, d)
            pooled2.append(jnp.maximum(m + b2_ref[...], 0.0))

    xfc = jnp.stack(pooled2, axis=0).reshape(25 * 32, tn)
    h = jnp.dot(fc1_ref[...], xfc, preferred_element_type=jnp.float32)
    h = jnp.maximum(h + fb1_ref[...], 0.0)
    h = jnp.dot(fc2_ref[...], h, preferred_element_type=jnp.float32)
    h = jnp.maximum(h + fb2_ref[...], 0.0)
    out = (jnp.dot(fc3_ref[...], h, preferred_element_type=jnp.float32)
           + fb3_ref[...])
    o_ref[...] = out.T


def kernel(conv1_w, conv1_b, conv2_w, conv2_b,
           fc1_w, fc1_b, fc2_w, fc2_b, fc3_w, fc3_b, x):
    N = x.shape[0]
    x2 = x.reshape(N, 28 * 28).T
    n_pad = -(-N // _TN) * _TN
    if n_pad != N:
        x2 = jnp.pad(x2, ((0, 0), (0, n_pad - N)))

    w1s = conv1_w.reshape(16, 25)
    w2f = conv2_w.reshape(32, 144)
    fc1_wp = fc1_w.reshape(120, 32, 25).transpose(0, 2, 1).reshape(120, 800)

    rep2 = lambda i: (0, 0)
    out = pl.pallas_call(
        _lenet_kernel,
        out_shape=jax.ShapeDtypeStruct((n_pad, 10), jnp.float32),
        grid=(n_pad // _TN,),
        in_specs=[
            pl.BlockSpec((28 * 28, _TN), lambda i: (0, i)),
            pl.BlockSpec(memory_space=pltpu.SMEM),
            pl.BlockSpec(memory_space=pltpu.SMEM),
            pl.BlockSpec((32, 144), rep2),
            pl.BlockSpec((32, 1), rep2),
            pl.BlockSpec((120, 800), rep2),
            pl.BlockSpec((120, 1), rep2),
            pl.BlockSpec((84, 120), rep2),
            pl.BlockSpec((84, 1), rep2),
            pl.BlockSpec((10, 84), rep2),
            pl.BlockSpec((10, 1), rep2),
        ],
        out_specs=pl.BlockSpec((_TN, 10), lambda i: (i, 0)),
        scratch_shapes=[pltpu.VMEM((16, 12, 12, _TN), jnp.float32)],
        compiler_params=pltpu.CompilerParams(
            dimension_semantics=("parallel",),
            vmem_limit_bytes=48 * 1024 * 1024,
        ),
    )(x2, w1s, conv1_b, w2f, conv2_b.reshape(32, 1),
      fc1_wp, fc1_b.reshape(120, 1), fc2_w, fc2_b.reshape(84, 1),
      fc3_w, fc3_b.reshape(10, 1))
    return out[:N]


# speedup vs baseline: 27.7180x; 1.3265x over previous
import jax
import jax.numpy as jnp
from jax.experimental import pallas as pl
from jax.experimental.pallas import tpu as pltpu

_TN = 256


def _lenet_kernel(x_ref, w1_ref, b1_ref, w2_ref, b2_ref,
                  fc1_ref, fb1_ref, fc2_ref, fb2_ref, fc3_ref, fb3_ref,
                  o_ref, y1_ref):
    tn = x_ref.shape[2]

    for s in range(6):
        accs = [None] * 16
        for i in range(5):
            for j in range(5):
                xs = x_ref[4 * s + i: 4 * s + i + 4, j:j + 24, :]
                for co in range(16):
                    w = w1_ref[co, i * 5 + j]
                    accs[co] = (w * xs if accs[co] is None
                                else accs[co] + w * xs)
        for co in range(16):
            m = jnp.maximum(accs[co] + b1_ref[co], 0.0)
            r = m.reshape(2, 2, 24, tn)
            m = jnp.maximum(r[:, 0], r[:, 1])
            c = m.reshape(2, 12, 2, tn)
            y1_ref[co, 2 * s: 2 * s + 2] = jnp.maximum(c[:, :, 0], c[:, :, 1])

    w2 = w2_ref[...]
    pooled2 = []
    for yp in range(5):
        for xp in range(5):
            m = None
            for a in range(2):
                for b in range(2):
                    y, x = 2 * yp + a, 2 * xp + b
                    patch = y1_ref[:, y:y + 3, x:x + 3, :].reshape(144, tn)
                    d = jnp.dot(w2, patch, preferred_element_type=jnp.float32)
                    m = d if m is None else jnp.maximum(m, d)
            pooled2.append(jnp.maximum(m + b2_ref[...], 0.0))

    xfc = jnp.stack(pooled2, axis=0).reshape(25 * 32, tn)
    h = jnp.dot(fc1_ref[...], xfc, preferred_element_type=jnp.float32)
    h = jnp.maximum(h + fb1_ref[...], 0.0)
    h = jnp.dot(fc2_ref[...], h, preferred_element_type=jnp.float32)
    h = jnp.maximum(h + fb2_ref[...], 0.0)
    out = (jnp.dot(fc3_ref[...], h, preferred_element_type=jnp.float32)
           + fb3_ref[...])
    o_ref[...] = out.T


def kernel(conv1_w, conv1_b, conv2_w, conv2_b,
           fc1_w, fc1_b, fc2_w, fc2_b, fc3_w, fc3_b, x):
    N = x.shape[0]
    x2 = x.reshape(N, 28 * 28).T
    n_pad = -(-N // _TN) * _TN
    if n_pad != N:
        x2 = jnp.pad(x2, ((0, 0), (0, n_pad - N)))
    x2 = x2.reshape(28, 28, n_pad)

    w1s = conv1_w.reshape(16, 25)
    w2f = conv2_w.reshape(32, 144)
    fc1_wp = fc1_w.reshape(120, 32, 25).transpose(0, 2, 1).reshape(120, 800)

    rep2 = lambda i: (0, 0)
    out = pl.pallas_call(
        _lenet_kernel,
        out_shape=jax.ShapeDtypeStruct((n_pad, 10), jnp.float32),
        grid=(n_pad // _TN,),
        in_specs=[
            pl.BlockSpec((28, 28, _TN), lambda i: (0, 0, i)),
            pl.BlockSpec(memory_space=pltpu.SMEM),
            pl.BlockSpec(memory_space=pltpu.SMEM),
            pl.BlockSpec((32, 144), rep2),
            pl.BlockSpec((32, 1), rep2),
            pl.BlockSpec((120, 800), rep2),
            pl.BlockSpec((120, 1), rep2),
            pl.BlockSpec((84, 120), rep2),
            pl.BlockSpec((84, 1), rep2),
            pl.BlockSpec((10, 84), rep2),
            pl.BlockSpec((10, 1), rep2),
        ],
        out_specs=pl.BlockSpec((_TN, 10), lambda i: (i, 0)),
        scratch_shapes=[pltpu.VMEM((16, 12, 12, _TN), jnp.float32)],
        compiler_params=pltpu.CompilerParams(
            dimension_semantics=("parallel",),
            vmem_limit_bytes=48 * 1024 * 1024,
        ),
    )(x2, w1s, conv1_b, w2f, conv2_b.reshape(32, 1),
      fc1_wp, fc1_b.reshape(120, 1), fc2_w, fc2_b.reshape(84, 1),
      fc3_w, fc3_b.reshape(10, 1))
    return out[:N]


# speedup vs baseline: 27.9427x; 1.0081x over previous
import jax
import jax.numpy as jnp
from jax.experimental import pallas as pl
from jax.experimental.pallas import tpu as pltpu

_TN = 512


def _lenet_kernel(x_ref, w1_ref, b1_ref, w2_ref, b2_ref,
                  fc1_ref, fb1_ref, fc2_ref, fb2_ref, fc3_ref, fb3_ref,
                  o_ref, y1_ref):
    tn = x_ref.shape[2]

    for s in range(6):
        accs = [None] * 16
        for i in range(5):
            for j in range(5):
                xs = x_ref[4 * s + i: 4 * s + i + 4, j:j + 24, :]
                for co in range(16):
                    w = w1_ref[co, i * 5 + j]
                    accs[co] = (w * xs if accs[co] is None
                                else accs[co] + w * xs)
        for co in range(16):
            m = jnp.maximum(accs[co] + b1_ref[co], 0.0)
            r = m.reshape(2, 2, 24, tn)
            m = jnp.maximum(r[:, 0], r[:, 1])
            c = m.reshape(2, 12, 2, tn)
            y1_ref[co, 2 * s: 2 * s + 2] = jnp.maximum(c[:, :, 0], c[:, :, 1])

    w2 = w2_ref[...]
    pooled2 = []
    for yp in range(5):
        for xp in range(5):
            m = None
            for a in range(2):
                for b in range(2):
                    y, x = 2 * yp + a, 2 * xp + b
                    patch = y1_ref[:, y:y + 3, x:x + 3, :].reshape(144, tn)
                    d = jnp.dot(w2, patch, preferred_element_type=jnp.float32)
                    m = d if m is None else jnp.maximum(m, d)
            pooled2.append(jnp.maximum(m + b2_ref[...], 0.0))

    xfc = jnp.stack(pooled2, axis=0).reshape(25 * 32, tn)
    h = jnp.dot(fc1_ref[...], xfc, preferred_element_type=jnp.float32)
    h = jnp.maximum(h + fb1_ref[...], 0.0)
    h = jnp.dot(fc2_ref[...], h, preferred_element_type=jnp.float32)
    h = jnp.maximum(h + fb2_ref[...], 0.0)
    out = (jnp.dot(fc3_ref[...], h, preferred_element_type=jnp.float32)
           + fb3_ref[...])
    o_ref[...] = out.T


def kernel(conv1_w, conv1_b, conv2_w, conv2_b,
           fc1_w, fc1_b, fc2_w, fc2_b, fc3_w, fc3_b, x):
    N = x.shape[0]
    x2 = x.reshape(N, 28 * 28).T
    n_pad = -(-N // _TN) * _TN
    if n_pad != N:
        x2 = jnp.pad(x2, ((0, 0), (0, n_pad - N)))
    x2 = x2.reshape(28, 28, n_pad)

    w1s = conv1_w.reshape(16, 25)
    w2f = conv2_w.reshape(32, 144)
    fc1_wp = fc1_w.reshape(120, 32, 25).transpose(0, 2, 1).reshape(120, 800)

    rep2 = lambda i: (0, 0)
    out = pl.pallas_call(
        _lenet_kernel,
        out_shape=jax.ShapeDtypeStruct((n_pad, 10), jnp.float32),
        grid=(n_pad // _TN,),
        in_specs=[
            pl.BlockSpec((28, 28, _TN), lambda i: (0, 0, i)),
            pl.BlockSpec(memory_space=pltpu.SMEM),
            pl.BlockSpec(memory_space=pltpu.SMEM),
            pl.BlockSpec((32, 144), rep2),
            pl.BlockSpec((32, 1), rep2),
            pl.BlockSpec((120, 800), rep2),
            pl.BlockSpec((120, 1), rep2),
            pl.BlockSpec((84, 120), rep2),
            pl.BlockSpec((84, 1), rep2),
            pl.BlockSpec((10, 84), rep2),
            pl.BlockSpec((10, 1), rep2),
        ],
        out_specs=pl.BlockSpec((_TN, 10), lambda i: (i, 0)),
        scratch_shapes=[pltpu.VMEM((16, 12, 12, _TN), jnp.float32)],
        compiler_params=pltpu.CompilerParams(
            dimension_semantics=("parallel",),
            vmem_limit_bytes=48 * 1024 * 1024,
        ),
    )(x2, w1s, conv1_b, w2f, conv2_b.reshape(32, 1),
      fc1_wp, fc1_b.reshape(120, 1), fc2_w, fc2_b.reshape(84, 1),
      fc3_w, fc3_b.reshape(10, 1))
    return out[:N]


# speedup vs baseline: 31.1037x; 1.1131x over previous
import jax
import jax.numpy as jnp
from jax.experimental import pallas as pl
from jax.experimental.pallas import tpu as pltpu

_TN = 512


def _lenet_kernel(x_ref, w1_ref, b1_ref, w2_ref, b2_ref,
                  fc1_ref, fb1_ref, fc2_ref, fb2_ref, fc3_ref, fb3_ref,
                  o_ref, y1_ref):
    tn = x_ref.shape[2]

    for s in range(6):
        accs = [None] * 16
        for i in range(5):
            for j in range(5):
                xs = x_ref[4 * s + i: 4 * s + i + 4, j:j + 24, :]
                for co in range(16):
                    w = w1_ref[co, i * 5 + j]
                    accs[co] = (w * xs if accs[co] is None
                                else accs[co] + w * xs)
        for co in range(16):
            m = jnp.maximum(accs[co] + b1_ref[co], 0.0)
            r = m.reshape(2, 2, 24, tn)
            m = jnp.maximum(r[:, 0], r[:, 1])
            c = m.reshape(2, 12, 2, tn)
            y1_ref[2 * s: 2 * s + 2, :, co, :] = jnp.maximum(c[:, :, 0], c[:, :, 1])

    w2 = w2_ref[...]
    pooled2 = []
    for yp in range(5):
        for xp in range(5):
            m = None
            for a in range(2):
                for b in range(2):
                    y, x = 2 * yp + a, 2 * xp + b
                    patch = y1_ref[y:y + 3, x:x + 3, :, :].reshape(144, tn)
                    d = jnp.dot(w2, patch, preferred_element_type=jnp.float32)
                    m = d if m is None else jnp.maximum(m, d)
            pooled2.append(jnp.maximum(m + b2_ref[...], 0.0))

    xfc = jnp.stack(pooled2, axis=0).reshape(25 * 32, tn)
    h = jnp.dot(fc1_ref[...], xfc, preferred_element_type=jnp.float32)
    h = jnp.maximum(h + fb1_ref[...], 0.0)
    h = jnp.dot(fc2_ref[...], h, preferred_element_type=jnp.float32)
    h = jnp.maximum(h + fb2_ref[...], 0.0)
    out = (jnp.dot(fc3_ref[...], h, preferred_element_type=jnp.float32)
           + fb3_ref[...])
    o_ref[...] = out.T


def kernel(conv1_w, conv1_b, conv2_w, conv2_b,
           fc1_w, fc1_b, fc2_w, fc2_b, fc3_w, fc3_b, x):
    N = x.shape[0]
    x2 = x.reshape(N, 28 * 28).T
    n_pad = -(-N // _TN) * _TN
    if n_pad != N:
        x2 = jnp.pad(x2, ((0, 0), (0, n_pad - N)))
    x2 = x2.reshape(28, 28, n_pad)

    w1s = conv1_w.reshape(16, 25)
    w2f = conv2_w.transpose(0, 2, 3, 1).reshape(32, 144)
    fc1_wp = fc1_w.reshape(120, 32, 25).transpose(0, 2, 1).reshape(120, 800)

    rep2 = lambda i: (0, 0)
    out = pl.pallas_call(
        _lenet_kernel,
        out_shape=jax.ShapeDtypeStruct((n_pad, 10), jnp.float32),
        grid=(n_pad // _TN,),
        in_specs=[
            pl.BlockSpec((28, 28, _TN), lambda i: (0, 0, i)),
            pl.BlockSpec(memory_space=pltpu.SMEM),
            pl.BlockSpec(memory_space=pltpu.SMEM),
            pl.BlockSpec((32, 144), rep2),
            pl.BlockSpec((32, 1), rep2),
            pl.BlockSpec((120, 800), rep2),
            pl.BlockSpec((120, 1), rep2),
            pl.BlockSpec((84, 120), rep2),
            pl.BlockSpec((84, 1), rep2),
            pl.BlockSpec((10, 84), rep2),
            pl.BlockSpec((10, 1), rep2),
        ],
        out_specs=pl.BlockSpec((_TN, 10), lambda i: (i, 0)),
        scratch_shapes=[pltpu.VMEM((12, 12, 16, _TN), jnp.float32)],
        compiler_params=pltpu.CompilerParams(
            dimension_semantics=("parallel",),
            vmem_limit_bytes=48 * 1024 * 1024,
        ),
    )(x2, w1s, conv1_b, w2f, conv2_b.reshape(32, 1),
      fc1_wp, fc1_b.reshape(120, 1), fc2_w, fc2_b.reshape(84, 1),
      fc3_w, fc3_b.reshape(10, 1))
    return out[:N]
